# baseline (speedup 1.0000x reference)
import functools

import jax
import jax.numpy as jnp
from jax import lax
from jax.experimental import pallas as pl
from jax.experimental.pallas import tpu as pltpu
from jax.experimental.pallas import tpu_sc as plsc

N = 10000
E = 320000
EMB = 128
OUT = 256
NDENSE = 3
NT = 12
RBF = 6

NC = 2
NS = 16
L = 16
CH = 128

ROWS = E // CH
WORKERS = NC * NS
RPW = ROWS // WORKERS
ROWS_MAIN = RPW * WORKERS
TAIL = ROWS - ROWS_MAIN
N_PAD = 10240
NPT = N_PAD // NS


def _sc_pool(x, rbf_flat, idx_flat, wrbf_flat, zeros):
    mesh = plsc.VectorSubcoreMesh(core_axis_name="c", subcore_axis_name="s")

    @functools.partial(
        pl.kernel,
        mesh=mesh,
        out_type=jax.ShapeDtypeStruct((NC, N_PAD, EMB), jnp.float32),
        scratch_types=[
            pltpu.VMEM((2, CH, EMB), jnp.float32),
            pltpu.VMEM((CH * RBF + L,), jnp.float32),
            pltpu.VMEM((CH * RBF + L,), jnp.float32),
            pltpu.VMEM((RBF * EMB,), jnp.float32),
            pltpu.VMEM((2, CH), jnp.int32),
            pltpu.VMEM_SHARED((N_PAD, EMB), jnp.float32),
            pltpu.SemaphoreType.DMA,
            pltpu.SemaphoreType.DMA,
        ],
    )
    def k(x_hbm, rbf_hbm, idx_hbm, w_hbm, z_hbm, out_hbm,
          xbuf, rbfbuf0, rbfbuf1, wbuf, idxbuf,
          acc, sem_ld0, sem_ld1):
        c = lax.axis_index("c")
        s = lax.axis_index("s")
        w = c * NS + s
        r_base = w * RPW
        sem_ld = (sem_ld0, sem_ld1)
        rbfb = (rbfbuf0, rbfbuf1)

        pltpu.sync_copy(z_hbm, acc.at[pl.ds(s * NPT, NPT)])
        pltpu.sync_copy(w_hbm, wbuf)
        plsc.subcore_barrier()

        def load_descs(r, b):
            return (
                pltpu.make_async_copy(
                    x_hbm.at[pl.ds(r * CH, CH)], xbuf.at[b], sem_ld[b]),
                pltpu.make_async_copy(
                    rbf_hbm.at[pl.ds(r * (CH * RBF), CH * RBF)],
                    rbfb[b].at[pl.ds(0, CH * RBF)], sem_ld[b]),
                pltpu.make_async_copy(
                    idx_hbm.at[pl.ds(r * CH, CH)], idxbuf.at[b], sem_ld[b]),
            )

        def start_loads(r, b):
            for d in load_descs(r, b):
                d.start()

        def wait_loads(r, b):
            for d in load_descs(r, b):
                d.wait()

        def compute(b):
            @plsc.parallel_loop(0, CH, 1, unroll=16)
            def edge_body(e):
                base = e * RBF
                coeffs = rbfb[b][pl.ds(base, L)]
                rr = [
                    jnp.full((L,), coeffs[j], jnp.float32)
                    for j in range(RBF)
                ]
                for blk in range(EMB // L):
                    g = rr[0] * wbuf[pl.ds(blk * L, L)]
                    for j in range(1, RBF):
                        g = g + rr[j] * wbuf[pl.ds(j * EMB + blk * L, L)]
                    xbuf[b, e, pl.ds(blk * L, L)] = (
                        g * xbuf[b, e, pl.ds(blk * L, L)])

        def half_iter(k_it, b, r):
            wait_loads(r, b)
            compute(b)
            pltpu.sync_copy(xbuf.at[b], acc.at[idxbuf.at[b]], add=True)

            @pl.when(k_it < (RPW // 2) - 1)
            def _():
                start_loads(r + 2, b)

        start_loads(r_base, 0)
        start_loads(r_base + 1, 1)

        def outer(k_it, carry):
            r0 = r_base + 2 * k_it
            half_iter(k_it, 0, r0)
            half_iter(k_it, 1, r0 + 1)
            return carry

        lax.fori_loop(0, RPW // 2, outer, 0)

        @pl.when(s < TAIL // NC)
        def _():
            r = ROWS_MAIN + c * (TAIL // NC) + s
            start_loads(r, 0)
            wait_loads(r, 0)
            compute(0)
            pltpu.sync_copy(xbuf.at[0], acc.at[idxbuf.at[0]], add=True)

        plsc.subcore_barrier()
        pltpu.sync_copy(acc.at[pl.ds(s * NPT, NPT)],
                        out_hbm.at[c, pl.ds(s * NPT, NPT)])

    return k(x, rbf_flat, idx_flat, wrbf_flat, zeros)


def _tc_mlp(partials, W_up, W_mlp, b_mlp, W_out):
    RB = 1000

    def body(p_ref, wu_ref, wm_ref, bm_ref, wo_ref, o_ref):
        p = p_ref[0] + p_ref[1]
        h = jnp.dot(p, wu_ref[...], preferred_element_type=jnp.float32)
        for i in range(NDENSE):
            v = jnp.dot(h, wm_ref[i], preferred_element_type=jnp.float32)
            v = v + bm_ref[i][None, :]
            h = v * jax.nn.sigmoid(v)
        o_ref[...] = jnp.dot(h, wo_ref[...],
                             preferred_element_type=jnp.float32)

    return pl.pallas_call(
        body,
        grid=(N // RB,),
        in_specs=[
            pl.BlockSpec((NC, RB, EMB), lambda i: (0, i, 0)),
            pl.BlockSpec((EMB, OUT), lambda i: (0, 0)),
            pl.BlockSpec((NDENSE, OUT, OUT), lambda i: (0, 0, 0)),
            pl.BlockSpec((NDENSE, OUT), lambda i: (0, 0)),
            pl.BlockSpec((OUT, NT), lambda i: (0, 0)),
        ],
        out_specs=pl.BlockSpec((RB, NT), lambda i: (i, 0)),
        out_shape=jax.ShapeDtypeStruct((N, NT), jnp.float32),
    )(partials, W_up, W_mlp, b_mlp, W_out)


def kernel(n_atoms, x, rbf, tensor_index, W_rbf, W_up, W_mlp, b_mlp, W_out):
    idx_flat = tensor_index.astype(jnp.int32)
    rbf_flat = rbf.reshape(E * RBF)
    wrbf_flat = W_rbf.reshape(RBF * EMB)
    zeros = jnp.zeros((NPT, EMB), jnp.float32)
    partials = _sc_pool(x, rbf_flat, idx_flat, wrbf_flat, zeros)
    return _tc_mlp(partials[:, :N, :], W_up, W_mlp, b_mlp, W_out)


# speedup vs baseline: 1.0252x; 1.0252x over previous
import functools

import jax
import jax.numpy as jnp
from jax import lax
from jax.experimental import pallas as pl
from jax.experimental.pallas import tpu as pltpu
from jax.experimental.pallas import tpu_sc as plsc

N = 10000
E = 320000
EMB = 128
OUT = 256
NDENSE = 3
NT = 12
RBF = 6

NC = 2
NS = 16
L = 16
CH = 128

ROWS = E // CH
WORKERS = NC * NS
RPW = ROWS // WORKERS
ROWS_MAIN = RPW * WORKERS
TAIL = ROWS - ROWS_MAIN
N_PAD = 10240
NPT = N_PAD // NS


def _sc_pool(x, rbf_flat, idx_flat, wrbf_flat, zeros):
    mesh = plsc.VectorSubcoreMesh(core_axis_name="c", subcore_axis_name="s")

    @functools.partial(
        pl.kernel,
        mesh=mesh,
        out_type=jax.ShapeDtypeStruct((NC, N_PAD, EMB), jnp.float32),
        scratch_types=[
            pltpu.VMEM((2, CH, EMB), jnp.float32),
            pltpu.VMEM((CH * RBF + L,), jnp.float32),
            pltpu.VMEM((CH * RBF + L,), jnp.float32),
            pltpu.VMEM((RBF * EMB,), jnp.float32),
            pltpu.VMEM((2, CH), jnp.int32),
            pltpu.VMEM_SHARED((N_PAD, EMB), jnp.float32),
            pltpu.SemaphoreType.DMA,
            pltpu.SemaphoreType.DMA,
        ],
    )
    def k(x_hbm, rbf_hbm, idx_hbm, w_hbm, z_hbm, out_hbm,
          xbuf, rbfbuf0, rbfbuf1, wbuf, idxbuf,
          acc, sem_ld0, sem_ld1):
        c = lax.axis_index("c")
        s = lax.axis_index("s")
        w = c * NS + s
        r_base = w * RPW
        sem_ld = (sem_ld0, sem_ld1)
        rbfb = (rbfbuf0, rbfbuf1)

        pltpu.sync_copy(z_hbm, acc.at[pl.ds(s * NPT, NPT)])
        pltpu.sync_copy(w_hbm, wbuf)
        plsc.subcore_barrier()

        def load_descs(r, b):
            return (
                pltpu.make_async_copy(
                    x_hbm.at[pl.ds(r * CH, CH)], xbuf.at[b], sem_ld[b]),
                pltpu.make_async_copy(
                    rbf_hbm.at[pl.ds(r * (CH * RBF), CH * RBF)],
                    rbfb[b].at[pl.ds(0, CH * RBF)], sem_ld[b]),
                pltpu.make_async_copy(
                    idx_hbm.at[pl.ds(r * CH, CH)], idxbuf.at[b], sem_ld[b]),
            )

        def start_loads(r, b):
            for d in load_descs(r, b):
                d.start()

        def wait_loads(r, b):
            for d in load_descs(r, b):
                d.wait()

        wv = [
            [wbuf[pl.ds(j * EMB + blk * L, L)] for j in range(RBF)]
            for blk in range(EMB // L)
        ]

        def compute(b):
            @plsc.parallel_loop(0, CH, 1, unroll=4)
            def edge_body(e):
                base = e * RBF
                coeffs = rbfb[b][pl.ds(base, L)]
                rr = [
                    jnp.full((L,), coeffs[j], jnp.float32)
                    for j in range(RBF)
                ]
                for blk in range(EMB // L):
                    g = rr[0] * wv[blk][0]
                    for j in range(1, RBF):
                        g = g + rr[j] * wv[blk][j]
                    xbuf[b, e, pl.ds(blk * L, L)] = (
                        g * xbuf[b, e, pl.ds(blk * L, L)])

        def half_iter(k_it, b, r):
            wait_loads(r, b)
            compute(b)
            pltpu.sync_copy(xbuf.at[b], acc.at[idxbuf.at[b]], add=True)

            @pl.when(k_it < (RPW // 2) - 1)
            def _():
                start_loads(r + 2, b)

        start_loads(r_base, 0)
        start_loads(r_base + 1, 1)

        def outer(k_it, carry):
            r0 = r_base + 2 * k_it
            half_iter(k_it, 0, r0)
            half_iter(k_it, 1, r0 + 1)
            return carry

        lax.fori_loop(0, RPW // 2, outer, 0)

        @pl.when(s < TAIL // NC)
        def _():
            r = ROWS_MAIN + c * (TAIL // NC) + s
            start_loads(r, 0)
            wait_loads(r, 0)
            compute(0)
            pltpu.sync_copy(xbuf.at[0], acc.at[idxbuf.at[0]], add=True)

        plsc.subcore_barrier()
        pltpu.sync_copy(acc.at[pl.ds(s * NPT, NPT)],
                        out_hbm.at[c, pl.ds(s * NPT, NPT)])

    return k(x, rbf_flat, idx_flat, wrbf_flat, zeros)


def _tc_mlp(partials, W_up, W_mlp, b_mlp, W_out):
    RB = 1000

    def body(p_ref, wu_ref, wm_ref, bm_ref, wo_ref, o_ref):
        p = p_ref[0] + p_ref[1]
        h = jnp.dot(p, wu_ref[...], preferred_element_type=jnp.float32)
        for i in range(NDENSE):
            v = jnp.dot(h, wm_ref[i], preferred_element_type=jnp.float32)
            v = v + bm_ref[i][None, :]
            h = v * jax.nn.sigmoid(v)
        o_ref[...] = jnp.dot(h, wo_ref[...],
                             preferred_element_type=jnp.float32)

    return pl.pallas_call(
        body,
        grid=(N // RB,),
        in_specs=[
            pl.BlockSpec((NC, RB, EMB), lambda i: (0, i, 0)),
            pl.BlockSpec((EMB, OUT), lambda i: (0, 0)),
            pl.BlockSpec((NDENSE, OUT, OUT), lambda i: (0, 0, 0)),
            pl.BlockSpec((NDENSE, OUT), lambda i: (0, 0)),
            pl.BlockSpec((OUT, NT), lambda i: (0, 0)),
        ],
        out_specs=pl.BlockSpec((RB, NT), lambda i: (i, 0)),
        out_shape=jax.ShapeDtypeStruct((N, NT), jnp.float32),
    )(partials, W_up, W_mlp, b_mlp, W_out)


def kernel(n_atoms, x, rbf, tensor_index, W_rbf, W_up, W_mlp, b_mlp, W_out):
    idx_flat = tensor_index.astype(jnp.int32)
    rbf_flat = rbf.reshape(E * RBF)
    wrbf_flat = W_rbf.reshape(RBF * EMB)
    zeros = jnp.zeros((NPT, EMB), jnp.float32)
    partials = _sc_pool(x, rbf_flat, idx_flat, wrbf_flat, zeros)
    return _tc_mlp(partials[:, :N, :], W_up, W_mlp, b_mlp, W_out)


# speedup vs baseline: 1.0370x; 1.0115x over previous
import functools

import jax
import jax.numpy as jnp
from jax import lax
from jax.experimental import pallas as pl
from jax.experimental.pallas import tpu as pltpu
from jax.experimental.pallas import tpu_sc as plsc

N = 10000
E = 320000
EMB = 128
OUT = 256
NDENSE = 3
NT = 12
RBF = 6

NC = 2
NS = 16
L = 16
CH = 128

ROWS = E // CH
WORKERS = NC * NS
RPW = ROWS // WORKERS
ROWS_MAIN = RPW * WORKERS
TAIL = ROWS - ROWS_MAIN
N_PAD = 10240
NPT = N_PAD // NS


def _sc_pool(x, rbf_flat, idx_flat, wrbf_flat, zeros):
    mesh = plsc.VectorSubcoreMesh(core_axis_name="c", subcore_axis_name="s")

    @functools.partial(
        pl.kernel,
        mesh=mesh,
        out_type=jax.ShapeDtypeStruct((NC, N_PAD, EMB), jnp.float32),
        scratch_types=[
            pltpu.VMEM((2, CH, EMB), jnp.float32),
            pltpu.VMEM((CH * RBF + L,), jnp.float32),
            pltpu.VMEM((CH * RBF + L,), jnp.float32),
            pltpu.VMEM((RBF * EMB,), jnp.float32),
            pltpu.VMEM((2, CH), jnp.int32),
            pltpu.VMEM_SHARED((N_PAD, EMB), jnp.float32),
            pltpu.SemaphoreType.DMA,
            pltpu.SemaphoreType.DMA,
        ],
    )
    def k(x_hbm, rbf_hbm, idx_hbm, w_hbm, z_hbm, out_hbm,
          xbuf, rbfbuf0, rbfbuf1, wbuf, idxbuf,
          acc, sem_ld0, sem_ld1):
        c = lax.axis_index("c")
        s = lax.axis_index("s")
        w = c * NS + s
        r_base = w * RPW
        sem_ld = (sem_ld0, sem_ld1)
        rbfb = (rbfbuf0, rbfbuf1)

        pltpu.sync_copy(z_hbm, acc.at[pl.ds(s * NPT, NPT)])
        pltpu.sync_copy(w_hbm, wbuf)
        plsc.subcore_barrier()

        def load_descs(r, b):
            return (
                pltpu.make_async_copy(
                    x_hbm.at[pl.ds(r * CH, CH)], xbuf.at[b], sem_ld[b]),
                pltpu.make_async_copy(
                    rbf_hbm.at[pl.ds(r * (CH * RBF), CH * RBF)],
                    rbfb[b].at[pl.ds(0, CH * RBF)], sem_ld[b]),
                pltpu.make_async_copy(
                    idx_hbm.at[pl.ds(r * CH, CH)], idxbuf.at[b], sem_ld[b]),
            )

        def start_loads(r, b):
            for d in load_descs(r, b):
                d.start()

        def wait_loads(r, b):
            for d in load_descs(r, b):
                d.wait()

        wv = [
            [wbuf[pl.ds(j * EMB + blk * L, L)] for j in range(RBF)]
            for blk in range(EMB // L)
        ]

        def compute(b):
            @plsc.parallel_loop(0, CH, 1, unroll=4)
            def edge_body(e):
                base = e * RBF
                coeffs = rbfb[b][pl.ds(base, L)]
                rr = [
                    jnp.full((L,), coeffs[j], jnp.float32)
                    for j in range(RBF)
                ]
                for blk in range(EMB // L):
                    g = rr[0] * wv[blk][0]
                    for j in range(1, RBF):
                        g = g + rr[j] * wv[blk][j]
                    xbuf[b, e, pl.ds(blk * L, L)] = (
                        g * xbuf[b, e, pl.ds(blk * L, L)])

        def half_iter(k_it, b, r):
            wait_loads(r, b)
            compute(b)
            pltpu.sync_copy(xbuf.at[b], acc.at[idxbuf.at[b]], add=True)

            @pl.when(k_it < (RPW // 2) - 1)
            def _():
                start_loads(r + 2, b)

        start_loads(r_base, 0)
        start_loads(r_base + 1, 1)

        def outer(k_it, carry):
            r0 = r_base + 2 * k_it
            half_iter(k_it, 0, r0)
            half_iter(k_it, 1, r0 + 1)
            return carry

        lax.fori_loop(0, RPW // 2, outer, 0)

        @pl.when(s < TAIL // NC)
        def _():
            r = ROWS_MAIN + c * (TAIL // NC) + s
            start_loads(r, 0)
            wait_loads(r, 0)
            compute(0)
            pltpu.sync_copy(xbuf.at[0], acc.at[idxbuf.at[0]], add=True)

        plsc.subcore_barrier()
        pltpu.sync_copy(acc.at[pl.ds(s * NPT, NPT)],
                        out_hbm.at[c, pl.ds(s * NPT, NPT)])

    return k(x, rbf_flat, idx_flat, wrbf_flat, zeros)


def _tc_mlp(partials, W_up, W_mlp, b_mlp, W_out):
    RB = 1000

    def body(p_ref, wu_ref, wm_ref, bm_ref, wo_ref, o_ref):
        p = p_ref[0] + p_ref[1]
        h = jnp.dot(p, wu_ref[...], preferred_element_type=jnp.float32)
        for i in range(NDENSE):
            v = jnp.dot(h, wm_ref[i], preferred_element_type=jnp.float32)
            v = v + bm_ref[i][None, :]
            h = v * jax.nn.sigmoid(v)
        o_ref[...] = jnp.dot(h, wo_ref[...],
                             preferred_element_type=jnp.float32)

    return pl.pallas_call(
        body,
        grid=(N // RB,),
        in_specs=[
            pl.BlockSpec((NC, RB, EMB), lambda i: (0, i, 0)),
            pl.BlockSpec((EMB, OUT), lambda i: (0, 0)),
            pl.BlockSpec((NDENSE, OUT, OUT), lambda i: (0, 0, 0)),
            pl.BlockSpec((NDENSE, OUT), lambda i: (0, 0)),
            pl.BlockSpec((OUT, NT), lambda i: (0, 0)),
        ],
        out_specs=pl.BlockSpec((RB, NT), lambda i: (i, 0)),
        out_shape=jax.ShapeDtypeStruct((N, NT), jnp.float32),
    )(partials, W_up, W_mlp, b_mlp, W_out)


def kernel(n_atoms, x, rbf, tensor_index, W_rbf, W_up, W_mlp, b_mlp, W_out):
    idx_flat = tensor_index.astype(jnp.int32)
    rbf_flat = rbf.reshape(E * RBF)
    wrbf_flat = W_rbf.reshape(RBF * EMB)
    zeros = jnp.zeros((NPT, EMB), jnp.float32)
    partials = _sc_pool(x, rbf_flat, idx_flat, wrbf_flat, zeros)
    return _tc_mlp(partials, W_up, W_mlp, b_mlp, W_out)
